# baseline (speedup 1.0000x reference)
import jax
import jax.numpy as jnp
from jax import lax
from jax.experimental import pallas as pl
from jax.experimental.pallas import tpu as pltpu
from jax.experimental.pallas import tpu_sc as plsc

N_NODES = 10000
D_FEAT = 256
HALF = D_FEAT // 2
NUM_SC = 2
NUM_TEC = 16
CHUNK = 128

ACC_ROWS = 10016
ZERO_ROWS = ACC_ROWS // NUM_TEC
OUT_ROWS = 624
TAIL_ROWS = N_NODES - NUM_TEC * OUT_ROWS


def _sc_kernel(n_chunks):
    assert n_chunks % 2 == 0

    def body(xs_hbm, idx_hbm, zer_hbm, out_hbm,
             idx0, idx1, rows_v, acc, zsem, isem0, isem1, gsem):
        idx_v = (idx0, idx1)
        isem = (isem0, isem1)
        c = lax.axis_index("c")
        s = lax.axis_index("s")
        ci0 = (c * NUM_TEC + s) * n_chunks

        pltpu.async_copy(
            zer_hbm, acc.at[pl.ds(s * ZERO_ROWS, ZERO_ROWS)], zsem).wait()
        plsc.subcore_barrier()

        def istart(g, b):
            pltpu.async_copy(idx_hbm.at[ci0 + g], idx_v[b], isem[b])

        def iwait(b):
            pltpu.make_async_copy(idx_hbm.at[0], idx_v[b], isem[b]).wait()

        istart(0, 0)

        def outer(o, carry):
            for b in range(2):
                g = 2 * o + b
                iwait(b)
                pltpu.async_copy(xs_hbm.at[idx_v[b].at[0]], rows_v,
                                 gsem).wait()
                istart(g + 1, 1 - b)
                pltpu.sync_copy(rows_v, acc.at[idx_v[b].at[1]], add=True)
            return carry

        lax.fori_loop(0, n_chunks // 2, outer, 0)
        iwait(0)
        plsc.subcore_barrier()

        pltpu.sync_copy(
            acc.at[pl.ds(s * OUT_ROWS, OUT_ROWS)],
            out_hbm.at[pl.ds(s * OUT_ROWS, OUT_ROWS), pl.ds(c * HALF, HALF)])

        @pl.when(s == NUM_TEC - 1)
        def _tail():
            r0 = NUM_TEC * OUT_ROWS
            pltpu.sync_copy(
                acc.at[pl.ds(r0, TAIL_ROWS)],
                out_hbm.at[pl.ds(r0, TAIL_ROWS), pl.ds(c * HALF, HALF)])

    mesh = plsc.VectorSubcoreMesh(core_axis_name="c", subcore_axis_name="s")
    return pl.kernel(
        body,
        out_type=jax.ShapeDtypeStruct((N_NODES, D_FEAT), jnp.float32),
        mesh=mesh,
        scratch_types=[
            pltpu.VMEM((2, CHUNK), jnp.int32),
            pltpu.VMEM((2, CHUNK), jnp.int32),
            pltpu.VMEM((CHUNK, HALF), jnp.float32),
            pltpu.VMEM_SHARED((ACC_ROWS, HALF), jnp.float32),
            pltpu.SemaphoreType.DMA,
            pltpu.SemaphoreType.DMA,
            pltpu.SemaphoreType.DMA,
            pltpu.SemaphoreType.DMA,
        ],
    )


@jax.jit
def kernel(x, up_index, down_index):
    n_edges = up_index.shape[1] + down_index.shape[1]
    align = NUM_TEC * CHUNK * 2
    e_pad = ((n_edges + align - 1) // align) * align
    n_chunks = e_pad // (NUM_TEC * CHUNK)
    pad = e_pad - n_edges

    src = jnp.concatenate(
        [up_index[0], down_index[0], jnp.zeros((pad,), up_index.dtype)]
    ).astype(jnp.int32)
    dst = jnp.concatenate(
        [up_index[1], down_index[1],
         jnp.full((pad,), N_NODES, up_index.dtype)]
    ).astype(jnp.int32)
    src_all = jnp.concatenate(
        [src, src + N_NODES, jnp.zeros((CHUNK,), jnp.int32)]
    ).reshape(-1, 1, CHUNK)
    dst_all = jnp.concatenate(
        [dst, dst, jnp.full((CHUNK,), N_NODES, jnp.int32)]
    ).reshape(-1, 1, CHUNK)
    idx_all = jnp.concatenate([src_all, dst_all], axis=1)
    xs = jnp.concatenate([x[:, :HALF], x[:, HALF:]], axis=0)
    zer = jnp.zeros((ZERO_ROWS, HALF), jnp.float32)

    return _sc_kernel(n_chunks)(xs, idx_all, zer)


# speedup vs baseline: 1.0424x; 1.0424x over previous
import jax
import jax.numpy as jnp
from jax import lax
from jax.experimental import pallas as pl
from jax.experimental.pallas import tpu as pltpu
from jax.experimental.pallas import tpu_sc as plsc

N_NODES = 10000
D_FEAT = 256
HALF = D_FEAT // 2
NUM_SC = 2
NUM_TEC = 16
CHUNK = 128

ACC_ROWS = 10016
ZERO_ROWS = ACC_ROWS // NUM_TEC
OUT_ROWS = 624
TAIL_ROWS = N_NODES - NUM_TEC * OUT_ROWS


def _sc_kernel(n_chunks):
    def body(xs_hbm, idx_hbm, zer_hbm, out_hbm,
             idx_v, rows_v, acc, zsem, gsem):
        c = lax.axis_index("c")
        s = lax.axis_index("s")
        ci0 = (c * NUM_TEC + s) * n_chunks

        pltpu.async_copy(
            zer_hbm, acc.at[pl.ds(s * ZERO_ROWS, ZERO_ROWS)], zsem).wait()
        plsc.subcore_barrier()

        def chunk(g, carry):
            pltpu.sync_copy(idx_hbm.at[ci0 + g], idx_v)
            pltpu.async_copy(xs_hbm.at[idx_v.at[0]], rows_v, gsem).wait()
            pltpu.sync_copy(rows_v, acc.at[idx_v.at[1]], add=True)
            return carry

        lax.fori_loop(0, n_chunks, chunk, 0)
        plsc.subcore_barrier()

        pltpu.sync_copy(
            acc.at[pl.ds(s * OUT_ROWS, OUT_ROWS)],
            out_hbm.at[pl.ds(s * OUT_ROWS, OUT_ROWS), pl.ds(c * HALF, HALF)])

        @pl.when(s == NUM_TEC - 1)
        def _tail():
            r0 = NUM_TEC * OUT_ROWS
            pltpu.sync_copy(
                acc.at[pl.ds(r0, TAIL_ROWS)],
                out_hbm.at[pl.ds(r0, TAIL_ROWS), pl.ds(c * HALF, HALF)])

    mesh = plsc.VectorSubcoreMesh(core_axis_name="c", subcore_axis_name="s")
    return pl.kernel(
        body,
        out_type=jax.ShapeDtypeStruct((N_NODES, D_FEAT), jnp.float32),
        mesh=mesh,
        scratch_types=[
            pltpu.VMEM((2, CHUNK), jnp.int32),
            pltpu.VMEM((CHUNK, HALF), jnp.float32),
            pltpu.VMEM_SHARED((ACC_ROWS, HALF), jnp.float32),
            pltpu.SemaphoreType.DMA,
            pltpu.SemaphoreType.DMA,
        ],
    )


@jax.jit
def kernel(x, up_index, down_index):
    n_edges = up_index.shape[1] + down_index.shape[1]
    align = NUM_TEC * CHUNK
    e_pad = ((n_edges + align - 1) // align) * align
    n_chunks = e_pad // align
    pad = e_pad - n_edges

    src = jnp.concatenate(
        [up_index[0], down_index[0], jnp.zeros((pad,), up_index.dtype)]
    ).astype(jnp.int32)
    dst = jnp.concatenate(
        [up_index[1], down_index[1],
         jnp.full((pad,), N_NODES, up_index.dtype)]
    ).astype(jnp.int32)
    src_all = jnp.concatenate([src, src + N_NODES]).reshape(-1, 1, CHUNK)
    dst_all = jnp.concatenate([dst, dst]).reshape(-1, 1, CHUNK)
    idx_all = jnp.concatenate([src_all, dst_all], axis=1)
    xs = jnp.concatenate([x[:, :HALF], x[:, HALF:]], axis=0)
    zer = jnp.zeros((ZERO_ROWS, HALF), jnp.float32)

    return _sc_kernel(n_chunks)(xs, idx_all, zer)
